# baseline (speedup 1.0000x reference)
import jax
import jax.numpy as jnp
from jax.experimental import pallas as pl
from jax.experimental.pallas import tpu as pltpu

AUG_THRESHOLD = 1024
TL = 2048
BIG = 2**30


def _aug_body(lens_ref, ptab_ref, x_ref, o_ref):
    b = pl.program_id(1)
    ln = lens_ref[b]
    lim = jnp.where(ln > AUG_THRESHOLD, ln, 0)
    mask = ptab_ref[...] < lim
    o_ref[...] = jnp.where(mask, 0.0, x_ref[...])


def kernel(sequences, seq_lens):
    B, L, D = sequences.shape
    pos = jnp.arange(L, dtype=jnp.int32)
    ptab = jnp.where(pos % 10 == 0, pos, BIG)[None, :, None]
    grid = (L // TL, B)
    out = pl.pallas_call(
        _aug_body,
        grid_spec=pltpu.PrefetchScalarGridSpec(
            num_scalar_prefetch=1,
            grid=grid,
            in_specs=[
                pl.BlockSpec((1, TL, 1), lambda t, b, lens: (0, t, 0)),
                pl.BlockSpec((1, TL, D), lambda t, b, lens: (b, t, 0)),
            ],
            out_specs=pl.BlockSpec((1, TL, D), lambda t, b, lens: (b, t, 0)),
        ),
        out_shape=jax.ShapeDtypeStruct((B, L, D), sequences.dtype),
        compiler_params=pltpu.CompilerParams(
            dimension_semantics=("parallel", "parallel"),
        ),
    )(seq_lens, ptab, sequences)
    return out, seq_lens


# speedup vs baseline: 1.4888x; 1.4888x over previous
import jax
import jax.numpy as jnp
from jax.experimental import pallas as pl
from jax.experimental.pallas import tpu as pltpu

AUG_THRESHOLD = 1024
BR = 4
BIG = 2**30


def _aug_body(lens_ref, ptab_ref, x_ref, o_ref):
    g = pl.program_id(0)
    ptab = ptab_ref[...]
    for j in range(BR):
        ln = lens_ref[g * BR + j]
        lim = jnp.where(ln > AUG_THRESHOLD, ln, 0)
        o_ref[j, :, :] = jnp.where(ptab[0] < lim, 0.0, x_ref[j, :, :])


def kernel(sequences, seq_lens):
    B, L, D = sequences.shape
    pos = jnp.arange(L, dtype=jnp.int32)
    ptab = jnp.where(pos % 10 == 0, pos, BIG)[None, :, None]
    grid = (B // BR,)
    out = pl.pallas_call(
        _aug_body,
        grid_spec=pltpu.PrefetchScalarGridSpec(
            num_scalar_prefetch=1,
            grid=grid,
            in_specs=[
                pl.BlockSpec((1, L, 1), lambda g, lens: (0, 0, 0)),
                pl.BlockSpec((BR, L, D), lambda g, lens: (g, 0, 0)),
            ],
            out_specs=pl.BlockSpec((BR, L, D), lambda g, lens: (g, 0, 0)),
        ),
        out_shape=jax.ShapeDtypeStruct((B, L, D), sequences.dtype),
        compiler_params=pltpu.CompilerParams(
            dimension_semantics=("parallel",),
        ),
    )(seq_lens, ptab, sequences)
    return out, seq_lens
